# Initial kernel scaffold; baseline (speedup 1.0000x reference)
import jax
import jax.numpy as jnp
from jax.experimental import pallas as pl


def kernel(x, token_table, position_table):
    raise NotImplementedError("write your pallas kernel here")


# speedup vs baseline: 1.2345x; 1.2345x over previous
import functools

import jax
import jax.numpy as jnp
from jax import lax
from jax.experimental import pallas as pl
from jax.experimental.pallas import tpu as pltpu
from jax.experimental.pallas import tpu_sc as plsc

NUM_VOCAB = 1000000
MAXLEN = 200
EMBED_DIM = 32
BATCH = 4096
SEQ = 200

NC = 2
NS = 16
NW = NC * NS
B = BATCH * SEQ
BPW = B // NW
CH = 200
NCHUNK = BPW // CH
HALF = 100
LANES = 16


def _emb_body(x_hbm, tok_hbm, pos_hbm, out_hbm, idx_v, buf_v, pos_v, sem):
    c = lax.axis_index("c")
    s = lax.axis_index("s")
    wid = s * NC + c

    pltpu.sync_copy(pos_hbm, pos_v)
    pltpu.sync_copy(x_hbm.at[pl.ds(wid * 2 * NCHUNK, 2 * NCHUNK)], idx_v)

    @pl.loop(0, NCHUNK)
    def _(k):
        cp0 = pltpu.async_copy(
            tok_hbm.at[idx_v.at[2 * k]], buf_v.at[pl.ds(0, HALF)], sem)
        cp1 = pltpu.async_copy(
            tok_hbm.at[idx_v.at[2 * k + 1]], buf_v.at[pl.ds(HALF, HALF)], sem)
        cp0.wait()
        cp1.wait()

        @pl.loop(0, CH)
        def _(r):
            buf_v[r, pl.ds(0, LANES)] = (
                buf_v[r, pl.ds(0, LANES)] + pos_v[r, pl.ds(0, LANES)])
            buf_v[r, pl.ds(LANES, LANES)] = (
                buf_v[r, pl.ds(LANES, LANES)] + pos_v[r, pl.ds(LANES, LANES)])

        row0 = (wid * NCHUNK + k) * CH
        pltpu.sync_copy(buf_v, out_hbm.at[pl.ds(row0, CH)])


def kernel(x, token_table, position_table):
    xf = x.reshape(B // HALF, HALF).astype(jnp.int32)
    mesh = plsc.VectorSubcoreMesh(core_axis_name="c", subcore_axis_name="s")
    run = pl.kernel(
        _emb_body,
        out_type=jax.ShapeDtypeStruct((B, EMBED_DIM), jnp.float32),
        mesh=mesh,
        scratch_types=[
            pltpu.VMEM((2 * NCHUNK, HALF), jnp.int32),
            pltpu.VMEM((CH, EMBED_DIM), jnp.float32),
            pltpu.VMEM((MAXLEN, EMBED_DIM), jnp.float32),
            pltpu.SemaphoreType.DMA,
        ],
        compiler_params=pltpu.CompilerParams(use_tc_tiling_on_sc=False),
    )
    out = run(xf, token_table, position_table)
    return out.reshape(BATCH, SEQ, EMBED_DIM)


# speedup vs baseline: 1.4628x; 1.1850x over previous
import jax
import jax.numpy as jnp
from jax import lax
from jax.experimental import pallas as pl
from jax.experimental.pallas import tpu as pltpu
from jax.experimental.pallas import tpu_sc as plsc

NUM_VOCAB = 1000000
MAXLEN = 200
EMBED_DIM = 32
BATCH = 4096
SEQ = 200

NC = 2
NS = 16
NW = NC * NS
B = BATCH * SEQ
BPW = B // NW
CH = 200
NCHUNK = BPW // CH
HALF = 100
LANES = 16
NBUF = 4


def _emb_body(x_hbm, tok_hbm, pos_hbm, out_hbm,
              idx_v, pos_v, gbufs, wbufs, gsems, wsems):
    c = lax.axis_index("c")
    s = lax.axis_index("s")
    wid = s * NC + c

    pltpu.sync_copy(pos_hbm, pos_v)
    pltpu.sync_copy(x_hbm.at[pl.ds(wid * 2 * NCHUNK, 2 * NCHUNK)], idx_v)

    def start_gather(k, b):
        pltpu.async_copy(
            tok_hbm.at[idx_v.at[2 * k]], gbufs[b].at[pl.ds(0, HALF)],
            gsems[b])
        pltpu.async_copy(
            tok_hbm.at[idx_v.at[2 * k + 1]], gbufs[b].at[pl.ds(HALF, HALF)],
            gsems[b])

    def wait_gather(b):
        pltpu.make_async_copy(
            tok_hbm.at[pl.ds(0, HALF)], gbufs[b].at[pl.ds(0, HALF)],
            gsems[b]).wait()
        pltpu.make_async_copy(
            tok_hbm.at[pl.ds(0, HALF)], gbufs[b].at[pl.ds(HALF, HALF)],
            gsems[b]).wait()

    def start_wb(k, b):
        row0 = (wid * NCHUNK + k) * CH
        pltpu.async_copy(wbufs[b], out_hbm.at[pl.ds(row0, CH)], wsems[b])

    def wait_wb(b):
        pltpu.make_async_copy(
            wbufs[b], out_hbm.at[pl.ds(0, CH)], wsems[b]).wait()

    for b in range(NBUF):
        start_gather(b, b)

    @pl.loop(0, NCHUNK, step=NBUF)
    def _(g):
        for b in range(NBUF):
            k = g + b
            wait_gather(b)

            @pl.when(g > 0)
            def _():
                wait_wb(b)

            gbuf, wbuf = gbufs[b], wbufs[b]

            @pl.loop(0, CH)
            def _(r):
                wbuf[r, pl.ds(0, LANES)] = (
                    gbuf[r, pl.ds(0, LANES)] + pos_v[r, pl.ds(0, LANES)])
                wbuf[r, pl.ds(LANES, LANES)] = (
                    gbuf[r, pl.ds(LANES, LANES)]
                    + pos_v[r, pl.ds(LANES, LANES)])

            @pl.when(g < NCHUNK - NBUF)
            def _():
                start_gather(k + NBUF, b)

            start_wb(k, b)

    for b in range(NBUF):
        wait_wb(b)


def kernel(x, token_table, position_table):
    xf = x.reshape(B // HALF, HALF).astype(jnp.int32)
    mesh = plsc.VectorSubcoreMesh(core_axis_name="c", subcore_axis_name="s")
    run = pl.kernel(
        _emb_body,
        out_type=jax.ShapeDtypeStruct((B, EMBED_DIM), jnp.float32),
        mesh=mesh,
        scratch_types=[
            pltpu.VMEM((2 * NCHUNK, HALF), jnp.int32),
            pltpu.VMEM((MAXLEN, EMBED_DIM), jnp.float32),
            [pltpu.VMEM((CH, EMBED_DIM), jnp.float32) for _ in range(NBUF)],
            [pltpu.VMEM((CH, EMBED_DIM), jnp.float32) for _ in range(NBUF)],
            [pltpu.SemaphoreType.DMA for _ in range(NBUF)],
            [pltpu.SemaphoreType.DMA for _ in range(NBUF)],
        ],
        compiler_params=pltpu.CompilerParams(use_tc_tiling_on_sc=False),
    )
    out = run(xf, token_table, position_table)
    return out.reshape(BATCH, SEQ, EMBED_DIM)
